# baseline (speedup 1.0000x reference)
import functools

import jax
import jax.numpy as jnp
from jax import lax
from jax.experimental import pallas as pl
from jax.experimental.pallas import tpu as pltpu
from jax.experimental.pallas import tpu_sc as plsc

L = 16
NC = 2
NS = 16
NW = NC * NS
SG = 8
QW = 4
NT = 1 << QW
NP = 4


def _make_sc_call(B, CTX, M, LOCAL):
    BW = B // NW
    NG = BW // SG
    NQ = CTX // QW
    MQ = M // NP
    MBQ = MQ // L
    TSTRIDE = NQ * MQ

    def body(ctx_hbm, pp0_hbm, pp1_hbm, pp2_hbm, pp3_hbm, ip_hbm, out_hbm,
             ctx_v, tq_v, ppA_v, ppB_v, ip_v, part_v, out_v, semA, semB):
        wid = lax.axis_index("c") * NS + lax.axis_index("s")
        base = wid * BW
        pp_hbms = [pp0_hbm, pp1_hbm, pp2_hbm, pp3_hbm]
        bufs = [ppA_v, ppB_v]
        sems = [semA, semB]

        first = pltpu.async_copy(pp_hbms[0], bufs[0], sems[0])
        pltpu.sync_copy(ctx_hbm.at[pl.ds(base * CTX, BW * CTX)], ctx_v)
        pltpu.sync_copy(ip_hbm, ip_v)
        lane_iota = lax.iota(jnp.int32, L)
        iota4 = lane_iota * QW

        cvecs = [[iota4 + (qc * L * QW + i) for i in range(QW)]
                 for qc in range(NQ // L)]

        def pack_body(s2, carry):
            for u in range(2):
                s = s2 * 2 + u
                rowvec = jnp.broadcast_to(s * CTX, (L,))
                for qc in range(NQ // L):
                    g = [plsc.load_gather(ctx_v, [rowvec + cvecs[qc][i]])
                         for i in range(QW)]
                    t = ((g[0] * 2 + g[1]) * 2 + g[2]) * 2 + g[3]
                    tq_v[pl.ds(s * NQ + qc * L, L)] = t
            return carry

        lax.fori_loop(0, BW // 2, pack_body, 0)

        pending = first
        for p in range(NP):
            pending.wait()
            if p + 1 < NP:
                pending = pltpu.async_copy(
                    pp_hbms[p + 1], bufs[(p + 1) % 2], sems[(p + 1) % 2])
            pp_v = bufs[p % 2]
            ip_rows = [[ip_v[pl.ds(l * M + p * MQ + j * L, L)]
                        for j in range(MBQ)] for l in range(LOCAL)]

            def group_body(g, carry):
                r0 = g * SG

                def chunk_body(qc, acc):
                    q0 = qc * L
                    tqrows = [tq_v[pl.ds((r0 + s) * NQ + q0, L)]
                              for s in range(SG)]
                    acc = list(acc)
                    for k in range(L):
                        qvec = lane_iota + (q0 + k) * MQ
                        for s in range(SG):
                            tb = jnp.broadcast_to(tqrows[s][k], (L,))
                            ib = tb * TSTRIDE + qvec
                            for j in range(MBQ):
                                row = plsc.load_gather(pp_v, [ib + j * L])
                                acc[s * MBQ + j] = acc[s * MBQ + j] * row
                    return tuple(acc)

                init = tuple(jnp.full((L,), 1.0, jnp.float32)
                             for _ in range(SG * MBQ))
                acc = lax.fori_loop(0, NQ // L, chunk_body, init)

                outvec = jnp.zeros((L,), jnp.float32)
                for s in range(SG):
                    for l in range(LOCAL):
                        v = acc[s * MBQ] * ip_rows[l][0]
                        for j in range(1, MBQ):
                            v = v + acc[s * MBQ + j] * ip_rows[l][j]
                        prow = ((r0 + s) * LOCAL + l) * L
                        if p == 0:
                            part_v[pl.ds(prow, L)] = v
                        elif p < NP - 1:
                            part_v[pl.ds(prow, L)] = (
                                part_v[pl.ds(prow, L)] + v)
                        else:
                            v = v + part_v[pl.ds(prow, L)]
                            for sh in (8, 4, 2, 1):
                                perm = lane_iota ^ sh
                                v = v + v.at[perm].get(
                                    mode="promise_in_bounds")
                            outvec = jnp.where(
                                lane_iota == s * LOCAL + l, v, outvec)
                if p == NP - 1:
                    out_v[pl.ds(g * L, L)] = outvec
                return carry

            lax.fori_loop(0, NG, group_body, 0)

        pltpu.sync_copy(out_v, out_hbm.at[pl.ds(base * LOCAL, BW * LOCAL)])

    mesh = plsc.VectorSubcoreMesh(core_axis_name="c", subcore_axis_name="s")
    return pl.kernel(
        body,
        mesh=mesh,
        compiler_params=pltpu.CompilerParams(needs_layout_passes=False),
        out_type=jax.ShapeDtypeStruct((B * LOCAL,), jnp.float32),
        scratch_types=[
            pltpu.VMEM((BW * CTX,), jnp.int32),
            pltpu.VMEM((BW * NQ,), jnp.int32),
            pltpu.VMEM((NT * NQ * MQ,), jnp.float32),
            pltpu.VMEM((NT * NQ * MQ,), jnp.float32),
            pltpu.VMEM((LOCAL * M,), jnp.float32),
            pltpu.VMEM((BW * LOCAL * L,), jnp.float32),
            pltpu.VMEM((BW * LOCAL,), jnp.float32),
            pltpu.SemaphoreType.DMA,
            pltpu.SemaphoreType.DMA,
        ],
    )


def kernel(context, context_param, inputs_param):
    LOCAL_N, M, CTX = context_param.shape
    B = context.shape[0]
    NQ = CTX // QW
    MQ = M // NP

    cpT = jnp.transpose(context_param, (0, 2, 1))
    gq = cpT.reshape(LOCAL_N, NQ, QW, M)
    t_idx = jnp.arange(NT)
    pp = (gq[(t_idx >> 3) & 1, :, 0, :]
          * gq[(t_idx >> 2) & 1, :, 1, :]
          * gq[(t_idx >> 1) & 1, :, 2, :]
          * gq[t_idx & 1, :, 3, :])
    ppq = [pp[:, :, i * MQ:(i + 1) * MQ].reshape(-1) for i in range(NP)]

    call = _make_sc_call(B, CTX, M, LOCAL_N)
    out = call(context.astype(jnp.int32).reshape(-1), *ppq,
               inputs_param.astype(jnp.float32).reshape(-1))
    return out.reshape(B, LOCAL_N)


# speedup vs baseline: 1.0006x; 1.0006x over previous
import functools

import jax
import jax.numpy as jnp
from jax import lax
from jax.experimental import pallas as pl
from jax.experimental.pallas import tpu as pltpu
from jax.experimental.pallas import tpu_sc as plsc

L = 16
NC = 2
NS = 16
NW = NC * NS
SG = 4
QW = 4
NT = 1 << QW


def _make_sc_call(B, CTX, M, LOCAL):
    BW = B // NW
    NG = BW // SG
    NQ = CTX // QW
    MH = M // 2
    MBH = MH // L
    TSTRIDE = NQ * MH
    SPG = 2 * SG

    def body(ctx_hbm, ppa_hbm, ppb_hbm, ip_hbm, out_hbm,
             ctx_v, tq_v, pp_v, ip_v, out_v):
        wid = lax.axis_index("c") * NS + lax.axis_index("s")
        base = wid * BW
        pltpu.sync_copy(ctx_hbm.at[pl.ds(base, BW)], ctx_v)
        pltpu.sync_copy(ip_hbm, ip_v)
        lane_iota = lax.iota(jnp.int32, L)
        iota4 = lane_iota * QW
        rowhalf = lane_iota // LOCAL
        colpar = lane_iota % LOCAL

        cvecs = [[iota4 + (qc * L * QW + i) for i in range(QW)]
                 for qc in range(NQ // L)]

        def pack_body(s2, carry):
            for u in range(2):
                s = s2 * 2 + u
                rowvec = jnp.broadcast_to(s, (L,))
                for qc in range(NQ // L):
                    g = [plsc.load_gather(ctx_v, [rowvec, cvecs[qc][i]])
                         for i in range(QW)]
                    t = ((g[0] * 2 + g[1]) * 2 + g[2]) * 2 + g[3]
                    tq_v[pl.ds(s * NQ + qc * L, L)] = t
            return carry

        lax.fori_loop(0, BW // 2, pack_body, 0)

        for half in range(2):
            pltpu.sync_copy(ppa_hbm if half == 0 else ppb_hbm, pp_v)
            ip_rows = [[ip_v[l, pl.ds(half * MH + j * L, L)]
                        for j in range(MBH)] for l in range(LOCAL)]

            def group_body(g, carry):
                r0 = g * SG

                def chunk_body(qc, acc):
                    q0 = qc * L
                    tqrows = [tq_v[pl.ds((r0 + s) * NQ + q0, L)]
                              for s in range(SG)]
                    acc = list(acc)
                    for k in range(L):
                        qvec = lane_iota + (q0 + k) * MH
                        for s in range(SG):
                            tb = jnp.broadcast_to(tqrows[s][k], (L,))
                            ib = tb * TSTRIDE + qvec
                            for j in range(MBH):
                                row = plsc.load_gather(pp_v, [ib + j * L])
                                acc[s * MBH + j] = acc[s * MBH + j] * row
                    return tuple(acc)

                init = tuple(jnp.full((L,), 1.0, jnp.float32)
                             for _ in range(SG * MBH))
                acc = lax.fori_loop(0, NQ // L, chunk_body, init)

                carry = list(carry)
                for s in range(SG):
                    for l in range(LOCAL):
                        lane = ((g % (SPG // SG)) * SG + s) * LOCAL + l
                        v = acc[s * MBH] * ip_rows[l][0]
                        for j in range(1, MBH):
                            v = v + acc[s * MBH + j] * ip_rows[l][j]
                        for sh in (8, 4, 2, 1):
                            perm = lane_iota ^ sh
                            v = v + v.at[perm].get(mode="promise_in_bounds")
                        carry[l] = jnp.where(lane_iota == lane, v, carry[l])

                @pl.when(g % (SPG // SG) == (SPG // SG) - 1)
                def _():
                    rows = rowhalf + (g // (SPG // SG)) * (L // LOCAL)
                    vec = carry[0] + carry[1]
                    if half == 0:
                        plsc.store_scatter(out_v, [rows, colpar], vec)
                    else:
                        old = plsc.load_gather(out_v, [rows, colpar])
                        plsc.store_scatter(out_v, [rows, colpar], old + vec)

                return tuple(carry)

            zero = jnp.zeros((L,), jnp.float32)
            lax.fori_loop(0, NG, group_body, (zero,) * LOCAL)

        pltpu.sync_copy(out_v, out_hbm.at[pl.ds(base, BW)])

    mesh = plsc.VectorSubcoreMesh(core_axis_name="c", subcore_axis_name="s")
    return pl.kernel(
        body,
        mesh=mesh,
        compiler_params=pltpu.CompilerParams(needs_layout_passes=False),
        out_type=jax.ShapeDtypeStruct((B, LOCAL), jnp.float32),
        scratch_types=[
            pltpu.VMEM((BW, CTX), jnp.int32),
            pltpu.VMEM((BW * NQ,), jnp.int32),
            pltpu.VMEM((NT * NQ * MH,), jnp.float32),
            pltpu.VMEM((LOCAL, M), jnp.float32),
            pltpu.VMEM((BW, LOCAL), jnp.float32),
        ],
    )


def kernel(context, context_param, inputs_param):
    LOCAL_N, M, CTX = context_param.shape
    B = context.shape[0]
    NQ = CTX // QW

    cpT = jnp.transpose(context_param, (0, 2, 1))
    gq = cpT.reshape(LOCAL_N, NQ, QW, M)
    t_idx = jnp.arange(NT)
    pp = (gq[(t_idx >> 3) & 1, :, 0, :]
          * gq[(t_idx >> 2) & 1, :, 1, :]
          * gq[(t_idx >> 1) & 1, :, 2, :]
          * gq[t_idx & 1, :, 3, :])
    ppa = pp[:, :, : M // 2].reshape(-1)
    ppb = pp[:, :, M // 2:].reshape(-1)

    call = _make_sc_call(B, CTX, M, LOCAL_N)
    return call(context.astype(jnp.int32), ppa, ppb,
                inputs_param.astype(jnp.float32))


# speedup vs baseline: 1.0410x; 1.0404x over previous
import functools

import jax
import jax.numpy as jnp
from jax import lax
from jax.experimental import pallas as pl
from jax.experimental.pallas import tpu as pltpu
from jax.experimental.pallas import tpu_sc as plsc

L = 16
NC = 2
NS = 16
NW = NC * NS
SG = 4
QW = 4
NT = 1 << QW


def _make_sc_call(B, CTX, M, LOCAL):
    BW = B // NW
    NG = BW // SG
    NQ = CTX // QW
    MH = M // 2
    MBH = MH // L
    TSTRIDE = NQ * MH
    SPG = 2 * SG

    def body(ctx_hbm, ppa_hbm, ppb_hbm, ip_hbm, out_hbm,
             ctx_v, tq_v, pp_v, ip_v, out_v):
        wid = lax.axis_index("c") * NS + lax.axis_index("s")
        base = wid * BW
        pltpu.sync_copy(ctx_hbm.at[pl.ds(base, BW)], ctx_v)
        pltpu.sync_copy(ip_hbm, ip_v)
        lane_iota = lax.iota(jnp.int32, L)
        iota4 = lane_iota * QW

        cvecs = [[iota4 + (qc * L * QW + i) for i in range(QW)]
                 for qc in range(NQ // L)]

        def pack_body(s2, carry):
            for u in range(2):
                s = s2 * 2 + u
                rowvec = jnp.broadcast_to(s, (L,))
                for qc in range(NQ // L):
                    g = [plsc.load_gather(ctx_v, [rowvec, cvecs[qc][i]])
                         for i in range(QW)]
                    t = ((g[0] * 2 + g[1]) * 2 + g[2]) * 2 + g[3]
                    tq_v[s, pl.ds(qc * L, L)] = t
            return carry

        lax.fori_loop(0, BW // 2, pack_body, 0)

        for half in range(2):
            pltpu.sync_copy(ppa_hbm if half == 0 else ppb_hbm, pp_v)
            ip_rows = [[ip_v[l, pl.ds(half * MH + j * L, L)]
                        for j in range(MBH)] for l in range(LOCAL)]

            def group_body(g, carry):
                r0 = g * SG

                def chunk_body(qc, acc):
                    q0 = qc * L
                    tqrows = [tq_v[r0 + s, pl.ds(q0, L)] for s in range(SG)]
                    acc = list(acc)
                    for k in range(L):
                        qvec = lane_iota + (q0 + k) * MH
                        for s in range(SG):
                            tb = jnp.broadcast_to(tqrows[s][k], (L,))
                            ib = tb * TSTRIDE + qvec
                            for j in range(MBH):
                                row = plsc.load_gather(pp_v, [ib + j * L])
                                acc[s * MBH + j] = acc[s * MBH + j] * row
                    return tuple(acc)

                init = tuple(jnp.full((L,), 1.0, jnp.float32)
                             for _ in range(SG * MBH))
                acc = lax.fori_loop(0, NQ // L, chunk_body, init)

                carry = list(carry)
                for s in range(SG):
                    for l in range(LOCAL):
                        lane = ((g % (SPG // SG)) * SG + s) * LOCAL + l
                        v = acc[s * MBH] * ip_rows[l][0]
                        for j in range(1, MBH):
                            v = v + acc[s * MBH + j] * ip_rows[l][j]
                        for sh in (8, 4, 2, 1):
                            perm = lane_iota ^ sh
                            v = v + v.at[perm].get(mode="promise_in_bounds")
                        carry[l] = jnp.where(lane_iota == lane, v, carry[l])

                @pl.when(g % (SPG // SG) == (SPG // SG) - 1)
                def _():
                    col = (g // (SPG // SG)) * L
                    for l in range(LOCAL):
                        if half == 0:
                            out_v[pl.ds(col, L)] = carry[l] if l == 0 else (
                                out_v[pl.ds(col, L)] + carry[l])
                        else:
                            out_v[pl.ds(col, L)] = (
                                out_v[pl.ds(col, L)] + carry[l])

                return tuple(carry)

            zero = jnp.zeros((L,), jnp.float32)
            lax.fori_loop(0, NG, group_body, (zero,) * LOCAL)

        pltpu.sync_copy(out_v, out_hbm.at[pl.ds(base * LOCAL, BW * LOCAL)])

    mesh = plsc.VectorSubcoreMesh(core_axis_name="c", subcore_axis_name="s")
    return pl.kernel(
        body,
        mesh=mesh,
        compiler_params=pltpu.CompilerParams(needs_layout_passes=False),
        out_type=jax.ShapeDtypeStruct((B * LOCAL,), jnp.float32),
        scratch_types=[
            pltpu.VMEM((BW, CTX), jnp.int32),
            pltpu.VMEM((BW, NQ), jnp.int32),
            pltpu.VMEM((NT * NQ * MH,), jnp.float32),
            pltpu.VMEM((LOCAL, M), jnp.float32),
            pltpu.VMEM((BW * LOCAL,), jnp.float32),
        ],
    )


def kernel(context, context_param, inputs_param):
    LOCAL_N, M, CTX = context_param.shape
    B = context.shape[0]
    NQ = CTX // QW

    cpT = jnp.transpose(context_param, (0, 2, 1))
    gq = cpT.reshape(LOCAL_N, NQ, QW, M)
    t_idx = jnp.arange(NT)
    pp = (gq[(t_idx >> 3) & 1, :, 0, :]
          * gq[(t_idx >> 2) & 1, :, 1, :]
          * gq[(t_idx >> 1) & 1, :, 2, :]
          * gq[t_idx & 1, :, 3, :])
    ppa = pp[:, :, : M // 2].reshape(-1)
    ppb = pp[:, :, M // 2:].reshape(-1)

    call = _make_sc_call(B, CTX, M, LOCAL_N)
    out = call(context.astype(jnp.int32), ppa, ppb,
               inputs_param.astype(jnp.float32))
    return out.reshape(B, LOCAL_N)
